# baseline (speedup 1.0000x reference)
import functools

import jax
import jax.numpy as jnp
from jax.experimental import pallas as pl
from jax.experimental.pallas import tpu as pltpu

_N = 32768
_ROWS = 256
_COLS = 128
_K = 128
_P = 0.9


def _rbo_kernel(p_in, t_in, out_ref, p_buf, t_buf):
    p_buf[...] = p_in[...]
    t_buf[...] = t_in[...]
    flat_idx = (
        jax.lax.broadcasted_iota(jnp.int32, (_ROWS, _COLS), 0) * _COLS
        + jax.lax.broadcasted_iota(jnp.int32, (_ROWS, _COLS), 1)
    )
    neg_inf = jnp.float32(-jnp.inf)
    big = jnp.int32(_N)

    def body(i, carry):
        acc, w = carry
        t = t_buf[...]
        tm = jnp.max(t)
        t_idx = jnp.min(jnp.where(t == tm, flat_idx, big))
        t_buf[...] = jnp.where(flat_idx == t_idx, neg_inf, t)

        p = p_buf[...]
        pm = jnp.max(p)
        p_idx = jnp.min(jnp.where(p == pm, flat_idx, big))
        p_buf[...] = jnp.where(flat_idx == p_idx, neg_inf, p)

        acc = acc + jnp.where(t_idx == p_idx, w, jnp.float32(0.0))
        return acc, w * jnp.float32(_P)

    acc, _ = jax.lax.fori_loop(
        0, _K, body, (jnp.float32(0.0), jnp.float32(1.0 - _P))
    )
    out_ref[0, 0] = jnp.float32(1.0) - acc


@jax.jit
def kernel(predictions, targets):
    p2 = predictions.reshape(_ROWS, _COLS)
    t2 = targets.reshape(_ROWS, _COLS)
    out = pl.pallas_call(
        _rbo_kernel,
        out_shape=jax.ShapeDtypeStruct((1, 1), jnp.float32),
        in_specs=[
            pl.BlockSpec(memory_space=pltpu.VMEM),
            pl.BlockSpec(memory_space=pltpu.VMEM),
        ],
        out_specs=pl.BlockSpec(memory_space=pltpu.SMEM),
        scratch_shapes=[
            pltpu.VMEM((_ROWS, _COLS), jnp.float32),
            pltpu.VMEM((_ROWS, _COLS), jnp.float32),
        ],
    )(p2, t2)
    return out[0, 0]
